# Initial kernel scaffold; baseline (speedup 1.0000x reference)
import functools

import jax
import jax.numpy as jnp
from jax import lax
from jax.experimental import pallas as pl
from jax.experimental.pallas import tpu as pltpu
from jax.experimental.pallas import tpu_sc as plsc

F32 = jnp.float32
I32 = jnp.int32

NC = 2
NS = 16
NW = NC * NS
L = 16
SUB = 128

UPD_W1, UPD_B1, UPD_W2, UPD_B2 = 0, 32, 48, 64
MSG_W1, MSG_B1, MSG_W2, MSG_B2 = 65, 97, 113, 145
ED_W1, ED_B1, ED_W2, ED_B2 = 147, 163, 179, 195
EU_W1, EU_B1, EU_W2, EU_B2 = 196, 212, 228, 260
WP = 264


def _iota():
    return lax.iota(I32, L)


def _mlp_1in(a, wv, ow1, ob1, ow2, ob2, dout):
    acc = [jnp.zeros((L,), F32) for _ in range(dout)]
    for k in range(16):
        t = jnp.maximum(a * wv[ow1 + k] + wv[ob1 + k], 0.0)
        for j in range(dout):
            acc[j] = acc[j] + wv[ow2 + k * dout + j] * t
    return [jnp.maximum(acc[j] + wv[ob2 + j], 0.0) for j in range(dout)]


def _mlp_2in(x0, x1, wv, ow1, ob1, ow2, ob2, dout):
    acc = [jnp.zeros((L,), F32) for _ in range(dout)]
    for k in range(16):
        t = jnp.maximum(x0 * wv[ow1 + k] + x1 * wv[ow1 + 16 + k] + wv[ob1 + k], 0.0)
        for j in range(dout):
            acc[j] = acc[j] + wv[ow2 + k * dout + j] * t
    return [jnp.maximum(acc[j] + wv[ob2 + j], 0.0) for j in range(dout)]


def _make_edge_kernel(n_ue, n_ap, e, nuep, napp):
    epw = e // NW
    nfull = epw // SUB
    tail = epw - nfull * SUB
    assert tail % L == 0 and e % NW == 0
    gpt = nuep // NS
    apt = napp // NS

    mesh = plsc.VectorSubcoreMesh(core_axis_name="c", subcore_axis_name="s")

    @functools.partial(
        pl.kernel,
        out_type=(
            jax.ShapeDtypeStruct((NC, nuep, 2), F32),
            jax.ShapeDtypeStruct((NC, napp, 4), F32),
        ),
        mesh=mesh,
        scratch_types=[
            pltpu.VMEM_SHARED((nuep, 2), F32),
            pltpu.VMEM_SHARED((nuep, 2), F32),
            pltpu.VMEM_SHARED((napp, 4), F32),
            pltpu.VMEM((WP,), F32),
            pltpu.VMEM((gpt, 2), F32),
            pltpu.VMEM((gpt, 2), F32),
            pltpu.VMEM((apt, 4), F32),
            pltpu.VMEM((SUB,), I32),
            pltpu.VMEM((SUB,), I32),
            pltpu.VMEM((SUB,), I32),
            pltpu.VMEM((SUB, 2), F32),
            pltpu.VMEM((SUB, 2), F32),
            pltpu.VMEM((SUB, 2), F32),
            pltpu.VMEM((SUB, 4), F32),
            pltpu.VMEM((SUB, 2), F32),
        ],
    )
    def edge_kernel(eid_hbm, attrd_hbm, eiu_hbm, attru_hbm, xpad_hbm, wts_hbm,
                    z2_hbm, z4_hbm, r2i_hbm, r4i_hbm, tiue_hbm, tiap_hbm, zi_hbm,
                    ue_parts, ap_parts,
                    g_sp, ue_acc, ap_acc,
                    wv, xg_v, gout_v, zap_v,
                    dstd_v, srcu_v, dstu_v,
                    attrd_v, attru_v, rows2_v, rows4_v, grows_v):
        c = lax.axis_index("c")
        s = lax.axis_index("s")
        wid = c * NS + s
        iot = _iota()
        zc = jnp.zeros((L,), I32)
        oc = jnp.full((L,), 1, I32)

        pltpu.sync_copy(wts_hbm, wv)
        pltpu.sync_copy(r2i_hbm, rows2_v)
        pltpu.sync_copy(r4i_hbm, rows4_v)

        pltpu.sync_copy(z2_hbm, xg_v)
        pltpu.sync_copy(xg_v, ue_acc.at[pl.ds(s * gpt, gpt), :])
        pltpu.sync_copy(z4_hbm, zap_v)
        pltpu.sync_copy(zap_v, ap_acc.at[pl.ds(s * apt, apt), :])

        pltpu.sync_copy(xpad_hbm.at[pl.ds(s * gpt, gpt), :], xg_v)

        def g_body(i, carry):
            eidx = iot + i * L
            x0 = plsc.load_gather(xg_v, [eidx, zc])
            x1 = plsc.load_gather(xg_v, [eidx, oc])
            g0, g1 = _mlp_2in(x0, x1, wv, MSG_W1, MSG_B1, MSG_W2, MSG_B2, 2)
            plsc.store_scatter(gout_v, [eidx, zc], g0)
            plsc.store_scatter(gout_v, [eidx, oc], g1)
            return carry

        lax.fori_loop(0, gpt // L, g_body, 0)
        pltpu.sync_copy(gout_v, g_sp.at[pl.ds(s * gpt, gpt), :])

        plsc.subcore_barrier()

        base = wid * epw

        def down_groups(ngrp):
            for g in range(ngrp):
                eidx = iot + g * L
                a = plsc.load_gather(attrd_v, [eidx, zc])
                (m,) = _mlp_1in(a, wv, ED_W1, ED_B1, ED_W2, ED_B2, 1)
                plsc.store_scatter(rows2_v, [eidx, zc], m)

        def down_body(j, carry):
            off = base + j * SUB
            pltpu.sync_copy(eid_hbm.at[1, pl.ds(off, SUB)], dstd_v)
            pltpu.sync_copy(attrd_hbm.at[pl.ds(off, SUB), :], attrd_v)
            down_groups(SUB // L)
            pltpu.sync_copy(rows2_v, ue_acc.at[dstd_v], add=True)
            return carry

        lax.fori_loop(0, nfull, down_body, 0)

        if tail:
            toff = base + nfull * SUB
            pltpu.sync_copy(tiue_hbm, dstd_v)
            pltpu.sync_copy(eid_hbm.at[1, pl.ds(toff, tail)],
                            dstd_v.at[pl.ds(0, tail)])
            pltpu.sync_copy(attrd_hbm.at[pl.ds(toff, tail), :],
                            attrd_v.at[pl.ds(0, tail), :])
            down_groups(tail // L)
            pltpu.sync_copy(rows2_v, ue_acc.at[dstd_v], add=True)

        def up_groups(ngrp):
            for g in range(ngrp):
                eidx = iot + g * L
                a = plsc.load_gather(attru_v, [eidx, zc])
                h0, h1 = _mlp_1in(a, wv, EU_W1, EU_B1, EU_W2, EU_B2, 2)
                g0 = plsc.load_gather(grows_v, [eidx, zc])
                g1 = plsc.load_gather(grows_v, [eidx, oc])
                plsc.store_scatter(rows4_v, [eidx, zc], g0 + h0)
                plsc.store_scatter(rows4_v, [eidx, oc], g1 + h1)

        def up_body(j, carry):
            off = base + j * SUB
            pltpu.sync_copy(eiu_hbm.at[0, pl.ds(off, SUB)], srcu_v)
            pltpu.sync_copy(eiu_hbm.at[1, pl.ds(off, SUB)], dstu_v)
            pltpu.sync_copy(attru_hbm.at[pl.ds(off, SUB), :], attru_v)
            pltpu.sync_copy(g_sp.at[srcu_v], grows_v)
            up_groups(SUB // L)
            pltpu.sync_copy(rows4_v, ap_acc.at[dstu_v], add=True)
            return carry

        lax.fori_loop(0, nfull, up_body, 0)

        if tail:
            toff = base + nfull * SUB
            pltpu.sync_copy(zi_hbm, srcu_v)
            pltpu.sync_copy(tiap_hbm, dstu_v)
            pltpu.sync_copy(eiu_hbm.at[0, pl.ds(toff, tail)],
                            srcu_v.at[pl.ds(0, tail)])
            pltpu.sync_copy(eiu_hbm.at[1, pl.ds(toff, tail)],
                            dstu_v.at[pl.ds(0, tail)])
            pltpu.sync_copy(attru_hbm.at[pl.ds(toff, tail), :],
                            attru_v.at[pl.ds(0, tail), :])
            pltpu.sync_copy(g_sp.at[srcu_v], grows_v)
            up_groups(tail // L)
            pltpu.sync_copy(rows4_v, ap_acc.at[dstu_v], add=True)

        plsc.subcore_barrier()

        pltpu.sync_copy(ue_acc.at[pl.ds(s * gpt, gpt), :], xg_v)
        pltpu.sync_copy(xg_v, ue_parts.at[c, pl.ds(s * gpt, gpt), :])
        pltpu.sync_copy(ap_acc.at[pl.ds(s * apt, apt), :], zap_v)
        pltpu.sync_copy(zap_v, ap_parts.at[c, pl.ds(s * apt, apt), :])

    return edge_kernel


def _make_finalize_kernel(nuep, napp):
    upt = nuep // NW
    apt = napp // NW
    mesh = plsc.VectorSubcoreMesh(core_axis_name="c", subcore_axis_name="s")

    @functools.partial(
        pl.kernel,
        out_type=(
            jax.ShapeDtypeStruct((nuep, 2), F32),
            jax.ShapeDtypeStruct((napp, 2), F32),
        ),
        mesh=mesh,
        scratch_types=[
            pltpu.VMEM((WP,), F32),
            pltpu.VMEM((nuep // NW, 2), F32),
            pltpu.VMEM((nuep // NW, 2), F32),
            pltpu.VMEM((nuep // NW, 2), F32),
            pltpu.VMEM((nuep // NW, 2), F32),
            pltpu.VMEM((napp // NW, 4), F32),
            pltpu.VMEM((napp // NW, 4), F32),
            pltpu.VMEM((napp // NW, 2), F32),
        ],
    )
    def finalize_kernel(xpad_hbm, ue_parts, ap_parts, wts_hbm,
                        oue_hbm, oap_hbm,
                        wv, xb_v, p0_v, p1_v, ob_v, a0_v, a1_v, oa_v):
        c = lax.axis_index("c")
        s = lax.axis_index("s")
        wid = c * NS + s
        iot = _iota()
        zc = jnp.zeros((L,), I32)
        oc = jnp.full((L,), 1, I32)
        tc = jnp.full((L,), 2, I32)

        pltpu.sync_copy(wts_hbm, wv)

        ub = wid * upt
        pltpu.sync_copy(xpad_hbm.at[pl.ds(ub, upt), :], xb_v)
        pltpu.sync_copy(ue_parts.at[0, pl.ds(ub, upt), :], p0_v)
        pltpu.sync_copy(ue_parts.at[1, pl.ds(ub, upt), :], p1_v)

        def ue_body(i, carry):
            eidx = iot + i * L
            x0 = plsc.load_gather(xb_v, [eidx, zc])
            x1 = plsc.load_gather(xb_v, [eidx, oc])
            (r,) = _mlp_2in(x0, x1, wv, UPD_W1, UPD_B1, UPD_W2, UPD_B2, 1)
            su = plsc.load_gather(p0_v, [eidx, zc]) + plsc.load_gather(p1_v, [eidx, zc])
            cn = plsc.load_gather(p0_v, [eidx, oc]) + plsc.load_gather(p1_v, [eidx, oc])
            avg = su / jnp.maximum(cn, 1.0)
            plsc.store_scatter(ob_v, [eidx, zc], x0)
            plsc.store_scatter(ob_v, [eidx, oc], avg + r)
            return carry

        lax.fori_loop(0, upt // L, ue_body, 0)
        pltpu.sync_copy(ob_v, oue_hbm.at[pl.ds(ub, upt), :])

        ab = wid * apt
        pltpu.sync_copy(ap_parts.at[0, pl.ds(ab, apt), :], a0_v)
        pltpu.sync_copy(ap_parts.at[1, pl.ds(ab, apt), :], a1_v)

        def ap_body(i, carry):
            eidx = iot + i * L
            s0 = plsc.load_gather(a0_v, [eidx, zc]) + plsc.load_gather(a1_v, [eidx, zc])
            s1 = plsc.load_gather(a0_v, [eidx, oc]) + plsc.load_gather(a1_v, [eidx, oc])
            cn = plsc.load_gather(a0_v, [eidx, tc]) + plsc.load_gather(a1_v, [eidx, tc])
            d = jnp.maximum(cn, 1.0)
            plsc.store_scatter(oa_v, [eidx, zc], s0 / d)
            plsc.store_scatter(oa_v, [eidx, oc], s1 / d)
            return carry

        lax.fori_loop(0, apt // L, ap_body, 0)
        pltpu.sync_copy(oa_v, oap_hbm.at[pl.ds(ab, apt), :])

    return finalize_kernel


def _round_up(n, m):
    return (n + m - 1) // m * m


def kernel(x_ue, x_ap, edge_index_down, edge_attr_down, edge_index_up, edge_attr_up,
           upd_ue_w1, upd_ue_b1, upd_ue_w2, upd_ue_b2,
           msg_ue_w1, msg_ue_b1, msg_ue_w2, msg_ue_b2,
           edge_down_w1, edge_down_b1, edge_down_w2, edge_down_b2,
           edge_up_w1, edge_up_b1, edge_up_w2, edge_up_b2):
    n_ue = x_ue.shape[0]
    n_ap = x_ap.shape[0]
    e = edge_attr_down.shape[0]
    nuep = _round_up(n_ue + 1, NW * L)
    napp = _round_up(n_ap + 1, NW * L)

    xpad = jnp.concatenate(
        [x_ue, jnp.zeros((nuep - n_ue, 2), F32)], axis=0)
    wts = jnp.concatenate([
        upd_ue_w1.reshape(-1), upd_ue_b1, upd_ue_w2.reshape(-1), upd_ue_b2,
        msg_ue_w1.reshape(-1), msg_ue_b1, msg_ue_w2.reshape(-1), msg_ue_b2,
        edge_down_w1.reshape(-1), edge_down_b1, edge_down_w2.reshape(-1), edge_down_b2,
        edge_up_w1.reshape(-1), edge_up_b1, edge_up_w2.reshape(-1), edge_up_b2,
        jnp.zeros((2,), F32),
    ])

    z2 = jnp.zeros((nuep // NS, 2), F32)
    z4 = jnp.zeros((napp // NS, 4), F32)
    r2i = jnp.tile(jnp.array([[0.0, 1.0]], F32), (SUB, 1))
    r4i = jnp.tile(jnp.array([[0.0, 0.0, 1.0, 0.0]], F32), (SUB, 1))
    tiue = jnp.full((SUB,), n_ue, I32)
    tiap = jnp.full((SUB,), n_ap, I32)
    zi = jnp.zeros((SUB,), I32)

    edge_kernel = _make_edge_kernel(n_ue, n_ap, e, nuep, napp)
    ue_parts, ap_parts = edge_kernel(
        edge_index_down, edge_attr_down, edge_index_up, edge_attr_up,
        xpad, wts, z2, z4, r2i, r4i, tiue, tiap, zi)

    finalize_kernel = _make_finalize_kernel(nuep, napp)
    oue, oap = finalize_kernel(xpad, ue_parts, ap_parts, wts)

    return oue[:n_ue], oap[:n_ap], edge_attr_down, edge_attr_up


# speedup vs baseline: 2.8860x; 2.8860x over previous
import functools

import jax
import jax.numpy as jnp
from jax import lax
from jax.experimental import pallas as pl
from jax.experimental.pallas import tpu as pltpu
from jax.experimental.pallas import tpu_sc as plsc

F32 = jnp.float32
I32 = jnp.int32

NC = 2
NS = 16
NW = NC * NS
L = 16
SUB = 128

S_UPD_W1R0, S_UPD_W1R1, S_UPD_B1, S_UPD_W2C0, S_UPD_B2R0 = 0, 1, 2, 3, 4
S_MSG_W1R0, S_MSG_W1R1, S_MSG_B1, S_MSG_W2C0, S_MSG_W2C1, S_MSG_B2R0, S_MSG_B2R1 = 5, 6, 7, 8, 9, 10, 11
S_ED_W1, S_ED_B1, S_ED_W2C0, S_ED_B2R0 = 12, 13, 14, 15
S_EU_W1, S_EU_B1, S_EU_W2C0, S_EU_W2C1, S_EU_B2R0, S_EU_B2R1 = 16, 17, 18, 19, 20, 21
NSLOT = 22
WP = NSLOT * L


def _iota():
    return lax.iota(I32, L)


def _slot(wv, s):
    return wv[pl.ds(s * L, L)]


def _scalars(vec):
    return [vec[k] for k in range(L)]


def _mlp_1in(a, w1s, b1s, w2s, b2v):
    dout = len(b2v)
    acc = [jnp.zeros((L,), F32) for _ in range(dout)]
    for k in range(16):
        t = jnp.maximum(a * w1s[k] + b1s[k], 0.0)
        for j in range(dout):
            acc[j] = acc[j] + w2s[j][k] * t
    return [jnp.maximum(acc[j] + b2v[j], 0.0) for j in range(dout)]


def _mlp_2in(x0, x1, w1s0, w1s1, b1s, w2s, b2v):
    dout = len(b2v)
    acc = [jnp.zeros((L,), F32) for _ in range(dout)]
    for k in range(16):
        t = jnp.maximum(x0 * w1s0[k] + x1 * w1s1[k] + b1s[k], 0.0)
        for j in range(dout):
            acc[j] = acc[j] + w2s[j][k] * t
    return [jnp.maximum(acc[j] + b2v[j], 0.0) for j in range(dout)]


def _make_edge_kernel(n_ue, n_ap, e, nuep, napp):
    epw = e // NW
    nfull = epw // SUB
    tail = epw - nfull * SUB
    assert tail % L == 0 and e % NW == 0
    gpt = nuep // NS
    apt = napp // NS

    mesh = plsc.VectorSubcoreMesh(core_axis_name="c", subcore_axis_name="s")

    @functools.partial(
        pl.kernel,
        out_type=(
            jax.ShapeDtypeStruct((NC, 2, nuep), F32),
            jax.ShapeDtypeStruct((NC, 3, napp), F32),
        ),
        mesh=mesh,
        compiler_params=pltpu.CompilerParams(
            use_tc_tiling_on_sc=False, needs_layout_passes=False),
        scratch_types=[
            pltpu.VMEM_SHARED((nuep,), F32),
            pltpu.VMEM_SHARED((nuep,), F32),
            pltpu.VMEM_SHARED((nuep,), F32),
            pltpu.VMEM_SHARED((nuep,), F32),
            pltpu.VMEM_SHARED((napp,), F32),
            pltpu.VMEM_SHARED((napp,), F32),
            pltpu.VMEM_SHARED((napp,), F32),
            pltpu.VMEM((WP,), F32),
            pltpu.VMEM((gpt,), F32),
            pltpu.VMEM((2 * gpt,), F32),
            pltpu.VMEM((gpt,), F32),
            pltpu.VMEM((gpt,), F32),
            pltpu.VMEM((SUB,), I32),
            pltpu.VMEM((SUB,), I32),
            pltpu.VMEM((SUB,), I32),
            pltpu.VMEM((2 * SUB,), F32),
            pltpu.VMEM((2 * SUB,), F32),
            pltpu.VMEM((SUB,), F32),
            pltpu.VMEM((SUB,), F32),
            pltpu.VMEM((SUB,), F32),
            pltpu.VMEM((SUB,), F32),
            pltpu.VMEM((SUB,), F32),
            pltpu.VMEM((SUB,), F32),
        ],
    )
    def edge_kernel(dstd_hbm, attrdf_hbm, srcu_hbm, dstu_hbm, attruf_hbm,
                    xpadf_hbm, wts_hbm, zue_hbm, ones_hbm,
                    tiue_hbm, tiap_hbm, zi_hbm,
                    ue_parts, ap_parts,
                    g0_sp, g1_sp, ue_sum, ue_cnt, ap_s0, ap_s1, ap_cnt,
                    wv, zb_v, xg_v, g0b_v, g1b_v,
                    dstd_v, srcu_v, dstu_v,
                    attrdf_v, attruf_v, msgd_v, m0_v, m1_v,
                    gr0_v, gr1_v, ones_v):
        c = lax.axis_index("c")
        s = lax.axis_index("s")
        wid = c * NS + s
        iot = _iota()

        pltpu.sync_copy(wts_hbm, wv)
        pltpu.sync_copy(ones_hbm, ones_v)

        pltpu.sync_copy(zue_hbm, zb_v)
        pltpu.sync_copy(zb_v, ue_sum.at[pl.ds(s * gpt, gpt)])
        pltpu.sync_copy(zb_v, ue_cnt.at[pl.ds(s * gpt, gpt)])
        pltpu.sync_copy(zb_v.at[pl.ds(0, apt)], ap_s0.at[pl.ds(s * apt, apt)])
        pltpu.sync_copy(zb_v.at[pl.ds(0, apt)], ap_s1.at[pl.ds(s * apt, apt)])
        pltpu.sync_copy(zb_v.at[pl.ds(0, apt)], ap_cnt.at[pl.ds(s * apt, apt)])

        pltpu.sync_copy(xpadf_hbm.at[pl.ds(2 * s * gpt, 2 * gpt)], xg_v)

        msg_w1s0 = _scalars(_slot(wv, S_MSG_W1R0))
        msg_w1s1 = _scalars(_slot(wv, S_MSG_W1R1))
        msg_b1s = _scalars(_slot(wv, S_MSG_B1))
        msg_w2s = [_scalars(_slot(wv, S_MSG_W2C0)), _scalars(_slot(wv, S_MSG_W2C1))]
        msg_b2v = [_slot(wv, S_MSG_B2R0), _slot(wv, S_MSG_B2R1)]

        def g_body(i, carry):
            fi = iot * 2 + i * (2 * L)
            x0 = plsc.load_gather(xg_v, [fi])
            x1 = plsc.load_gather(xg_v, [fi + 1])
            g0, g1 = _mlp_2in(x0, x1, msg_w1s0, msg_w1s1, msg_b1s, msg_w2s, msg_b2v)
            g0b_v[pl.ds(i * L, L)] = g0
            g1b_v[pl.ds(i * L, L)] = g1
            return carry

        lax.fori_loop(0, gpt // L, g_body, 0)
        pltpu.sync_copy(g0b_v, g0_sp.at[pl.ds(s * gpt, gpt)])
        pltpu.sync_copy(g1b_v, g1_sp.at[pl.ds(s * gpt, gpt)])

        plsc.subcore_barrier()

        base = wid * epw

        ed_w1s = _scalars(_slot(wv, S_ED_W1))
        ed_b1s = _scalars(_slot(wv, S_ED_B1))
        ed_w2s = [_scalars(_slot(wv, S_ED_W2C0))]
        ed_b2v = [_slot(wv, S_ED_B2R0)]

        def down_groups(ngrp):
            for g in range(ngrp):
                fi = iot * 2 + g * (2 * L)
                a = plsc.load_gather(attrdf_v, [fi])
                (m,) = _mlp_1in(a, ed_w1s, ed_b1s, ed_w2s, ed_b2v)
                msgd_v[pl.ds(g * L, L)] = m

        def down_scatter():
            pltpu.sync_copy(msgd_v, ue_sum.at[dstd_v], add=True)
            pltpu.sync_copy(ones_v, ue_cnt.at[dstd_v], add=True)

        def down_body(j, carry):
            off = base + j * SUB
            pltpu.sync_copy(dstd_hbm.at[pl.ds(off, SUB)], dstd_v)
            pltpu.sync_copy(attrdf_hbm.at[pl.ds(2 * off, 2 * SUB)], attrdf_v)
            down_groups(SUB // L)
            down_scatter()
            return carry

        lax.fori_loop(0, nfull, down_body, 0)

        if tail:
            toff = base + nfull * SUB
            pltpu.sync_copy(tiue_hbm, dstd_v)
            pltpu.sync_copy(dstd_hbm.at[pl.ds(toff, tail)],
                            dstd_v.at[pl.ds(0, tail)])
            pltpu.sync_copy(attrdf_hbm.at[pl.ds(2 * toff, 2 * tail)],
                            attrdf_v.at[pl.ds(0, 2 * tail)])
            down_groups(tail // L)
            down_scatter()

        eu_w1s = _scalars(_slot(wv, S_EU_W1))
        eu_b1s = _scalars(_slot(wv, S_EU_B1))
        eu_w2s = [_scalars(_slot(wv, S_EU_W2C0)), _scalars(_slot(wv, S_EU_W2C1))]
        eu_b2v = [_slot(wv, S_EU_B2R0), _slot(wv, S_EU_B2R1)]

        def up_groups(ngrp):
            for g in range(ngrp):
                fi = iot * 2 + g * (2 * L)
                a = plsc.load_gather(attruf_v, [fi])
                h0, h1 = _mlp_1in(a, eu_w1s, eu_b1s, eu_w2s, eu_b2v)
                m0_v[pl.ds(g * L, L)] = gr0_v[pl.ds(g * L, L)] + h0
                m1_v[pl.ds(g * L, L)] = gr1_v[pl.ds(g * L, L)] + h1

        def up_scatter():
            pltpu.sync_copy(m0_v, ap_s0.at[dstu_v], add=True)
            pltpu.sync_copy(m1_v, ap_s1.at[dstu_v], add=True)
            pltpu.sync_copy(ones_v, ap_cnt.at[dstu_v], add=True)

        def up_body(j, carry):
            off = base + j * SUB
            pltpu.sync_copy(srcu_hbm.at[pl.ds(off, SUB)], srcu_v)
            pltpu.sync_copy(dstu_hbm.at[pl.ds(off, SUB)], dstu_v)
            pltpu.sync_copy(attruf_hbm.at[pl.ds(2 * off, 2 * SUB)], attruf_v)
            pltpu.sync_copy(g0_sp.at[srcu_v], gr0_v)
            pltpu.sync_copy(g1_sp.at[srcu_v], gr1_v)
            up_groups(SUB // L)
            up_scatter()
            return carry

        lax.fori_loop(0, nfull, up_body, 0)

        if tail:
            toff = base + nfull * SUB
            pltpu.sync_copy(zi_hbm, srcu_v)
            pltpu.sync_copy(tiap_hbm, dstu_v)
            pltpu.sync_copy(srcu_hbm.at[pl.ds(toff, tail)],
                            srcu_v.at[pl.ds(0, tail)])
            pltpu.sync_copy(dstu_hbm.at[pl.ds(toff, tail)],
                            dstu_v.at[pl.ds(0, tail)])
            pltpu.sync_copy(attruf_hbm.at[pl.ds(2 * toff, 2 * tail)],
                            attruf_v.at[pl.ds(0, 2 * tail)])
            pltpu.sync_copy(g0_sp.at[srcu_v], gr0_v)
            pltpu.sync_copy(g1_sp.at[srcu_v], gr1_v)
            up_groups(tail // L)
            up_scatter()

        plsc.subcore_barrier()

        pltpu.sync_copy(ue_sum.at[pl.ds(s * gpt, gpt)], zb_v)
        pltpu.sync_copy(zb_v, ue_parts.at[c, 0, pl.ds(s * gpt, gpt)])
        pltpu.sync_copy(ue_cnt.at[pl.ds(s * gpt, gpt)], zb_v)
        pltpu.sync_copy(zb_v, ue_parts.at[c, 1, pl.ds(s * gpt, gpt)])
        pltpu.sync_copy(ap_s0.at[pl.ds(s * apt, apt)], zb_v.at[pl.ds(0, apt)])
        pltpu.sync_copy(zb_v.at[pl.ds(0, apt)], ap_parts.at[c, 0, pl.ds(s * apt, apt)])
        pltpu.sync_copy(ap_s1.at[pl.ds(s * apt, apt)], zb_v.at[pl.ds(0, apt)])
        pltpu.sync_copy(zb_v.at[pl.ds(0, apt)], ap_parts.at[c, 1, pl.ds(s * apt, apt)])
        pltpu.sync_copy(ap_cnt.at[pl.ds(s * apt, apt)], zb_v.at[pl.ds(0, apt)])
        pltpu.sync_copy(zb_v.at[pl.ds(0, apt)], ap_parts.at[c, 2, pl.ds(s * apt, apt)])

    return edge_kernel


def _make_finalize_kernel(nuep, napp):
    upt = nuep // NW
    apt = napp // NW
    mesh = plsc.VectorSubcoreMesh(core_axis_name="c", subcore_axis_name="s")

    @functools.partial(
        pl.kernel,
        out_type=(
            jax.ShapeDtypeStruct((2 * nuep,), F32),
            jax.ShapeDtypeStruct((2 * napp,), F32),
        ),
        mesh=mesh,
        compiler_params=pltpu.CompilerParams(
            use_tc_tiling_on_sc=False, needs_layout_passes=False),
        scratch_types=[
            pltpu.VMEM((WP,), F32),
            pltpu.VMEM((2 * upt,), F32),
            pltpu.VMEM((upt,), F32),
            pltpu.VMEM((upt,), F32),
            pltpu.VMEM((upt,), F32),
            pltpu.VMEM((upt,), F32),
            pltpu.VMEM((2 * upt,), F32),
            pltpu.VMEM((apt,), F32),
            pltpu.VMEM((apt,), F32),
            pltpu.VMEM((apt,), F32),
            pltpu.VMEM((apt,), F32),
            pltpu.VMEM((apt,), F32),
            pltpu.VMEM((apt,), F32),
            pltpu.VMEM((2 * apt,), F32),
        ],
    )
    def finalize_kernel(xpadf_hbm, ue_parts, ap_parts, wts_hbm,
                        oue_hbm, oap_hbm,
                        wv, xb_v, s0_v, s1_v, c0_v, c1_v, ob_v,
                        as00_v, as01_v, as10_v, as11_v, ac0_v, ac1_v, oa_v):
        c = lax.axis_index("c")
        s = lax.axis_index("s")
        wid = c * NS + s
        iot = _iota()

        pltpu.sync_copy(wts_hbm, wv)

        ub = wid * upt
        pltpu.sync_copy(xpadf_hbm.at[pl.ds(2 * ub, 2 * upt)], xb_v)
        pltpu.sync_copy(ue_parts.at[0, 0, pl.ds(ub, upt)], s0_v)
        pltpu.sync_copy(ue_parts.at[1, 0, pl.ds(ub, upt)], s1_v)
        pltpu.sync_copy(ue_parts.at[0, 1, pl.ds(ub, upt)], c0_v)
        pltpu.sync_copy(ue_parts.at[1, 1, pl.ds(ub, upt)], c1_v)

        upd_w1s0 = _scalars(_slot(wv, S_UPD_W1R0))
        upd_w1s1 = _scalars(_slot(wv, S_UPD_W1R1))
        upd_b1s = _scalars(_slot(wv, S_UPD_B1))
        upd_w2s = [_scalars(_slot(wv, S_UPD_W2C0))]
        upd_b2v = [_slot(wv, S_UPD_B2R0)]

        def ue_body(i, carry):
            fi = iot * 2 + i * (2 * L)
            x0 = plsc.load_gather(xb_v, [fi])
            x1 = plsc.load_gather(xb_v, [fi + 1])
            (r,) = _mlp_2in(x0, x1, upd_w1s0, upd_w1s1, upd_b1s, upd_w2s, upd_b2v)
            su = s0_v[pl.ds(i * L, L)] + s1_v[pl.ds(i * L, L)]
            cn = c0_v[pl.ds(i * L, L)] + c1_v[pl.ds(i * L, L)]
            avg = su / jnp.maximum(cn, 1.0)
            plsc.store_scatter(ob_v, [fi], x0)
            plsc.store_scatter(ob_v, [fi + 1], avg + r)
            return carry

        lax.fori_loop(0, upt // L, ue_body, 0)
        pltpu.sync_copy(ob_v, oue_hbm.at[pl.ds(2 * ub, 2 * upt)])

        ab = wid * apt
        pltpu.sync_copy(ap_parts.at[0, 0, pl.ds(ab, apt)], as00_v)
        pltpu.sync_copy(ap_parts.at[1, 0, pl.ds(ab, apt)], as01_v)
        pltpu.sync_copy(ap_parts.at[0, 1, pl.ds(ab, apt)], as10_v)
        pltpu.sync_copy(ap_parts.at[1, 1, pl.ds(ab, apt)], as11_v)
        pltpu.sync_copy(ap_parts.at[0, 2, pl.ds(ab, apt)], ac0_v)
        pltpu.sync_copy(ap_parts.at[1, 2, pl.ds(ab, apt)], ac1_v)

        def ap_body(i, carry):
            fi = iot * 2 + i * (2 * L)
            s0 = as00_v[pl.ds(i * L, L)] + as01_v[pl.ds(i * L, L)]
            s1 = as10_v[pl.ds(i * L, L)] + as11_v[pl.ds(i * L, L)]
            cn = ac0_v[pl.ds(i * L, L)] + ac1_v[pl.ds(i * L, L)]
            d = jnp.maximum(cn, 1.0)
            plsc.store_scatter(oa_v, [fi], s0 / d)
            plsc.store_scatter(oa_v, [fi + 1], s1 / d)
            return carry

        lax.fori_loop(0, apt // L, ap_body, 0)
        pltpu.sync_copy(oa_v, oap_hbm.at[pl.ds(2 * ab, 2 * apt)])

    return finalize_kernel


def _round_up(n, m):
    return (n + m - 1) // m * m


def kernel(x_ue, x_ap, edge_index_down, edge_attr_down, edge_index_up, edge_attr_up,
           upd_ue_w1, upd_ue_b1, upd_ue_w2, upd_ue_b2,
           msg_ue_w1, msg_ue_b1, msg_ue_w2, msg_ue_b2,
           edge_down_w1, edge_down_b1, edge_down_w2, edge_down_b2,
           edge_up_w1, edge_up_b1, edge_up_w2, edge_up_b2):
    n_ue = x_ue.shape[0]
    n_ap = x_ap.shape[0]
    e = edge_attr_down.shape[0]
    nuep = _round_up(n_ue + 1, NW * L)
    napp = _round_up(n_ap + 1, NW * L)

    xpadf = jnp.concatenate(
        [x_ue, jnp.zeros((nuep - n_ue, 2), F32)], axis=0).reshape(-1)

    def rep(b):
        return jnp.full((L,), b, F32)

    wts = jnp.concatenate([
        upd_ue_w1[0], upd_ue_w1[1], upd_ue_b1, upd_ue_w2[:, 0], rep(upd_ue_b2[0]),
        msg_ue_w1[0], msg_ue_w1[1], msg_ue_b1, msg_ue_w2[:, 0], msg_ue_w2[:, 1],
        rep(msg_ue_b2[0]), rep(msg_ue_b2[1]),
        edge_down_w1[0], edge_down_b1, edge_down_w2[:, 0], rep(edge_down_b2[0]),
        edge_up_w1[0], edge_up_b1, edge_up_w2[:, 0], edge_up_w2[:, 1],
        rep(edge_up_b2[0]), rep(edge_up_b2[1]),
    ])

    zue = jnp.zeros((nuep // NS,), F32)
    ones = jnp.ones((SUB,), F32)
    tiue = jnp.full((SUB,), n_ue, I32)
    tiap = jnp.full((SUB,), n_ap, I32)
    zi = jnp.zeros((SUB,), I32)

    edge_kernel = _make_edge_kernel(n_ue, n_ap, e, nuep, napp)
    ue_parts, ap_parts = edge_kernel(
        edge_index_down[1], edge_attr_down.reshape(-1),
        edge_index_up[0], edge_index_up[1], edge_attr_up.reshape(-1),
        xpadf, wts, zue, ones, tiue, tiap, zi)

    finalize_kernel = _make_finalize_kernel(nuep, napp)
    oue, oap = finalize_kernel(xpadf, ue_parts, ap_parts, wts)

    return (oue.reshape(nuep, 2)[:n_ue], oap.reshape(napp, 2)[:n_ap],
            edge_attr_down, edge_attr_up)
